# baseline (speedup 1.0000x reference)
import functools

import jax
import jax.numpy as jnp
from jax import lax
from jax.experimental import pallas as pl
from jax.experimental.pallas import tpu as pltpu
from jax.experimental.pallas import tpu_sc as plsc

D_MODEL = 768
D_FF = 3072
E = 8
T = 2048
K = 2

M = 128
A = T * K
NB = A // M + E
P = NB * M

NW = 32
APW = A // NW
TPW = T // NW

_INV_SQRT2 = 0.7071067811865476
_H = D_MODEL // 2


def _gelu(h):
    return h * 0.5 * (1.0 + lax.erf(h * _INV_SQRT2))


def _pack_rows(y):
    def rnd(v):
        u = lax.bitcast_convert_type(v, jnp.uint32)
        r = u + jnp.uint32(0x7FFF) + ((u >> 16) & jnp.uint32(1))
        return r & jnp.uint32(0xFFFF0000)
    ul = rnd(y[:, :_H])
    uh = rnd(y[:, _H:])
    return lax.bitcast_convert_type((ul >> 16) | uh, jnp.int32)


def _unpack_rows(xi):
    u = lax.bitcast_convert_type(xi, jnp.uint32)
    lo = lax.bitcast_convert_type(u << 16, jnp.float32)
    hi = lax.bitcast_convert_type(u & jnp.uint32(0xFFFF0000), jnp.float32)
    return jnp.concatenate([lo, hi], axis=1)


def _routing_body(x_ref, gw_ref, pos_ref, w_ref, xpk_ref, be_ref, nv_ref):
    x = x_ref[...]
    xpk_ref[...] = _pack_rows(x)
    logits = lax.dot_general(
        x, gw_ref[...], (((1,), (1,)), ((), ())),
        preferred_element_type=jnp.float32)
    m = jnp.max(logits, axis=1, keepdims=True)
    p = jnp.exp(logits - m)
    p = p / jnp.sum(p, axis=1, keepdims=True)
    iota_e = jax.lax.broadcasted_iota(jnp.int32, (T, E), 1)
    m1 = jnp.max(p, axis=1, keepdims=True)
    i1 = jnp.min(jnp.where(p == m1, iota_e, E), axis=1, keepdims=True)
    p2 = jnp.where(iota_e == i1, -jnp.inf, p)
    m2 = jnp.max(p2, axis=1, keepdims=True)
    i2 = jnp.min(jnp.where(p2 == m2, iota_e, E), axis=1, keepdims=True)
    s = m1 + m2 + 1e-9
    w_ref[...] = jnp.concatenate([m1 / s, m2 / s], axis=1)

    onehot1 = (iota_e == i1).astype(jnp.float32)
    onehot2 = (iota_e == i2).astype(jnp.float32)

    r_iota = jax.lax.broadcasted_iota(jnp.int32, (T, T), 0)
    c_iota = jax.lax.broadcasted_iota(jnp.int32, (T, T), 1)
    tril = (r_iota > c_iota).astype(jnp.bfloat16)
    both = jnp.concatenate([onehot1, onehot2], axis=1).astype(jnp.bfloat16)
    ranks = jnp.dot(tril, both, preferred_element_type=jnp.float32)
    rank1 = ranks[:, :E]
    rank2 = ranks[:, E:]
    totals1 = jnp.sum(onehot1, axis=0, keepdims=True)
    totals2 = jnp.sum(onehot2, axis=0, keepdims=True)
    counts = totals1 + totals2

    nb = jnp.floor((counts + (M - 1)) / M)
    e_r = jax.lax.broadcasted_iota(jnp.int32, (E, E), 0)
    e_c = jax.lax.broadcasted_iota(jnp.int32, (E, E), 1)
    upper_strict = (e_r < e_c).astype(jnp.float32)
    bs = jnp.dot(nb, upper_strict, preferred_element_type=jnp.float32)
    cum_incl = bs + nb

    bs1 = jnp.sum(onehot1 * bs, axis=1, keepdims=True)
    bs2 = jnp.sum(onehot2 * bs, axis=1, keepdims=True)
    r1 = jnp.sum(onehot1 * rank1, axis=1, keepdims=True)
    r2 = jnp.sum(onehot2 * (rank2 + totals1), axis=1, keepdims=True)
    p1 = bs1 * M + r1
    p2_ = bs2 * M + r2
    pos_ref[...] = jnp.concatenate([p1, p2_], axis=0).astype(jnp.int32)

    b_iota = jax.lax.broadcasted_iota(jnp.int32, (NB, E), 0).astype(jnp.float32)
    be = jnp.sum((b_iota >= cum_incl).astype(jnp.float32), axis=1, keepdims=True)
    be_ref[...] = jnp.minimum(be, E - 1).astype(jnp.int32)
    nv_ref[...] = jnp.sum(nb, axis=1, keepdims=True).astype(jnp.int32)


@functools.lru_cache(maxsize=None)
def _sc_scatter_kernel():
    mesh = plsc.VectorSubcoreMesh(core_axis_name="c", subcore_axis_name="s")

    @functools.partial(
        pl.kernel,
        out_type=jax.ShapeDtypeStruct((P, D_MODEL // 2), jnp.int32),
        mesh=mesh,
        scratch_types=[
            pltpu.VMEM((APW,), jnp.int32),
            pltpu.VMEM((APW, D_MODEL // 2), jnp.int32),
            pltpu.SemaphoreType.DMA,
        ],
    )
    def k(x_hbm, pos_hbm, xs_hbm, pos_v, rows_v, sem):
        nc = 2
        wid = lax.axis_index("s") * nc + lax.axis_index("c")
        a0 = wid * APW
        t0 = lax.rem(a0, T)
        pltpu.sync_copy(pos_hbm.at[pl.ds(a0, APW)], pos_v)
        pltpu.sync_copy(x_hbm.at[pl.ds(t0, APW)], rows_v)
        pltpu.async_copy(rows_v, xs_hbm.at[pos_v], sem).wait()

    return k


def _sc_scatter_x(xf, pos_flat):
    return _sc_scatter_kernel()(xf, pos_flat)


@functools.lru_cache(maxsize=None)
def _sc_gather_kernel():
    mesh = plsc.VectorSubcoreMesh(core_axis_name="c", subcore_axis_name="s")

    @functools.partial(
        pl.kernel,
        out_type=(
            jax.ShapeDtypeStruct((T, D_MODEL // 2), jnp.int32),
            jax.ShapeDtypeStruct((T, D_MODEL // 2), jnp.int32),
        ),
        mesh=mesh,
        scratch_types=[
            pltpu.VMEM((TPW,), jnp.int32),
            pltpu.VMEM((TPW,), jnp.int32),
            pltpu.VMEM((TPW, D_MODEL // 2), jnp.int32),
            pltpu.VMEM((TPW, D_MODEL // 2), jnp.int32),
            pltpu.SemaphoreType.DMA,
            pltpu.SemaphoreType.DMA,
        ],
    )
    def k(ys_hbm, pos_hbm, ga_hbm, gb_hbm, p0_v, p1_v, a_v, b_v, s0, s1):
        nc = 2
        wid = lax.axis_index("s") * nc + lax.axis_index("c")
        t0 = wid * TPW
        pltpu.sync_copy(pos_hbm.at[pl.ds(t0, TPW)], p0_v)
        pltpu.sync_copy(pos_hbm.at[pl.ds(T + t0, TPW)], p1_v)
        ca = pltpu.async_copy(ys_hbm.at[p0_v], a_v, s0)
        cb = pltpu.async_copy(ys_hbm.at[p1_v], b_v, s1)
        ca.wait()
        cb.wait()
        pltpu.sync_copy(a_v, ga_hbm.at[pl.ds(t0, TPW)])
        pltpu.sync_copy(b_v, gb_hbm.at[pl.ds(t0, TPW)])

    return k


def _sc_gather_ys(ys, pos_flat):
    return _sc_gather_kernel()(ys, pos_flat)


def _mlp_body(be_ref, nv_ref, xs_ref, W1_ref, b1_ref, W2_ref, b2_ref, ys_ref):
    b = pl.program_id(0)

    @pl.when(b < nv_ref[0, 0])
    def _():
        xs = _unpack_rows(xs_ref[...])
        h = jnp.dot(xs, W1_ref[0], preferred_element_type=jnp.float32)
        h = _gelu(h + b1_ref[0])
        y = jnp.dot(h, W2_ref[0], preferred_element_type=jnp.float32) + b2_ref[0]
        ys_ref[...] = _pack_rows(y)


def _combine_body(w_ref, ga_ref, gb_ref, out_ref):
    w = w_ref[...]
    ga = _unpack_rows(ga_ref[...])
    gb = _unpack_rows(gb_ref[...])
    out_ref[...] = ga * w[:, 0:1] + gb * w[:, 1:2]


def kernel(x, gate_w, W1, b1, W2, b2):
    B, S, D = x.shape
    xf = x.reshape(S, D)

    pos_c, w2c, xpk, be2, nv2 = pl.pallas_call(
        _routing_body,
        out_shape=(
            jax.ShapeDtypeStruct((A, 1), jnp.int32),
            jax.ShapeDtypeStruct((T, 2), jnp.float32),
            jax.ShapeDtypeStruct((T, _H), jnp.int32),
            jax.ShapeDtypeStruct((NB, 1), jnp.int32),
            jax.ShapeDtypeStruct((1, 1), jnp.int32),
        ),
    )(xf, gate_w)

    pos_flat = pos_c.reshape(A)

    xs = _sc_scatter_x(xpk, pos_flat)

    ys = pl.pallas_call(
        _mlp_body,
        grid_spec=pltpu.PrefetchScalarGridSpec(
            num_scalar_prefetch=2,
            grid=(NB,),
            in_specs=[
                pl.BlockSpec((M, _H), lambda b, be, nv: (b, 0)),
                pl.BlockSpec((1, D_MODEL, D_FF),
                             lambda b, be, nv: (be[b, 0], 0, 0)),
                pl.BlockSpec((1, 1, D_FF), lambda b, be, nv: (be[b, 0], 0, 0)),
                pl.BlockSpec((1, D_FF, D_MODEL),
                             lambda b, be, nv: (be[b, 0], 0, 0)),
                pl.BlockSpec((1, 1, D_MODEL),
                             lambda b, be, nv: (be[b, 0], 0, 0)),
            ],
            out_specs=pl.BlockSpec((M, _H), lambda b, be, nv: (b, 0)),
        ),
        out_shape=jax.ShapeDtypeStruct((P, _H), jnp.int32),
        compiler_params=pltpu.CompilerParams(
            dimension_semantics=("arbitrary",),
        ),
    )(be2, nv2, xs, W1, b1.reshape(E, 1, D_FF), W2, b2.reshape(E, 1, D_MODEL))

    ga, gb = _sc_gather_ys(ys, pos_flat)

    out = pl.pallas_call(
        _combine_body,
        out_shape=jax.ShapeDtypeStruct((T, D_MODEL), jnp.float32),
    )(w2c, ga, gb)

    return out.reshape(B, S, D)


# speedup vs baseline: 1.0063x; 1.0063x over previous
import functools

import jax
import jax.numpy as jnp
from jax import lax
from jax.experimental import pallas as pl
from jax.experimental.pallas import tpu as pltpu
from jax.experimental.pallas import tpu_sc as plsc

D_MODEL = 768
D_FF = 3072
E = 8
T = 2048
K = 2

M = 128
A = T * K
NB = A // M + E
P = NB * M

NW = 32
APW = A // NW
TPW = T // NW

_INV_SQRT2 = 0.7071067811865476
_H = D_MODEL // 2


def _gelu(h):
    return h * 0.5 * (1.0 + lax.erf(h * _INV_SQRT2))


def _pack_rows(y):
    def rnd(v):
        u = lax.bitcast_convert_type(v, jnp.uint32)
        r = u + jnp.uint32(0x7FFF) + ((u >> 16) & jnp.uint32(1))
        return r & jnp.uint32(0xFFFF0000)
    ul = rnd(y[:, :_H])
    uh = rnd(y[:, _H:])
    return lax.bitcast_convert_type((ul >> 16) | uh, jnp.int32)


def _unpack_rows(xi):
    u = lax.bitcast_convert_type(xi, jnp.uint32)
    lo = lax.bitcast_convert_type(u << 16, jnp.float32)
    hi = lax.bitcast_convert_type(u & jnp.uint32(0xFFFF0000), jnp.float32)
    return jnp.concatenate([lo, hi], axis=1)


def _routing_body(x_ref, gw_ref, pos_ref, w_ref, xpk_ref, be_ref, nv_ref):
    x = x_ref[...]
    xpk_ref[...] = _pack_rows(x)
    logits = lax.dot_general(
        x, gw_ref[...], (((1,), (1,)), ((), ())),
        preferred_element_type=jnp.float32)
    m = jnp.max(logits, axis=1, keepdims=True)
    p = jnp.exp(logits - m)
    p = p / jnp.sum(p, axis=1, keepdims=True)
    iota_e = jax.lax.broadcasted_iota(jnp.int32, (T, E), 1)
    m1 = jnp.max(p, axis=1, keepdims=True)
    i1 = jnp.min(jnp.where(p == m1, iota_e, E), axis=1, keepdims=True)
    p2 = jnp.where(iota_e == i1, -jnp.inf, p)
    m2 = jnp.max(p2, axis=1, keepdims=True)
    i2 = jnp.min(jnp.where(p2 == m2, iota_e, E), axis=1, keepdims=True)
    s = m1 + m2 + 1e-9
    w_ref[...] = jnp.concatenate([m1 / s, m2 / s], axis=1)

    onehot1 = (iota_e == i1).astype(jnp.float32)
    onehot2 = (iota_e == i2).astype(jnp.float32)

    r_iota = jax.lax.broadcasted_iota(jnp.int32, (T, T), 0)
    c_iota = jax.lax.broadcasted_iota(jnp.int32, (T, T), 1)
    tril = (r_iota > c_iota).astype(jnp.bfloat16)
    both = jnp.concatenate([onehot1, onehot2], axis=1).astype(jnp.bfloat16)
    ranks = jnp.dot(tril, both, preferred_element_type=jnp.float32)
    rank1 = ranks[:, :E]
    rank2 = ranks[:, E:]
    totals1 = jnp.sum(onehot1, axis=0, keepdims=True)
    totals2 = jnp.sum(onehot2, axis=0, keepdims=True)
    counts = totals1 + totals2

    nb = jnp.floor((counts + (M - 1)) / M)
    e_r = jax.lax.broadcasted_iota(jnp.int32, (E, E), 0)
    e_c = jax.lax.broadcasted_iota(jnp.int32, (E, E), 1)
    upper_strict = (e_r < e_c).astype(jnp.float32)
    bs = jnp.dot(nb, upper_strict, preferred_element_type=jnp.float32)
    cum_incl = bs + nb

    bs1 = jnp.sum(onehot1 * bs, axis=1, keepdims=True)
    bs2 = jnp.sum(onehot2 * bs, axis=1, keepdims=True)
    r1 = jnp.sum(onehot1 * rank1, axis=1, keepdims=True)
    r2 = jnp.sum(onehot2 * (rank2 + totals1), axis=1, keepdims=True)
    p1 = bs1 * M + r1
    p2_ = bs2 * M + r2
    pos_ref[...] = jnp.concatenate([p1, p2_], axis=0).astype(jnp.int32)

    b_iota = jax.lax.broadcasted_iota(jnp.int32, (NB, E), 0).astype(jnp.float32)
    be = jnp.sum((b_iota >= cum_incl).astype(jnp.float32), axis=1, keepdims=True)
    be_ref[...] = jnp.minimum(be, E - 1).astype(jnp.int32)
    nv_ref[...] = jnp.sum(nb, axis=1, keepdims=True).astype(jnp.int32)


@functools.lru_cache(maxsize=None)
def _sc_scatter_kernel():
    mesh = plsc.VectorSubcoreMesh(core_axis_name="c", subcore_axis_name="s")

    @functools.partial(
        pl.kernel,
        out_type=jax.ShapeDtypeStruct((P, D_MODEL // 2), jnp.int32),
        mesh=mesh,
        scratch_types=[
            pltpu.VMEM((APW,), jnp.int32),
            pltpu.VMEM((APW, D_MODEL // 2), jnp.int32),
            pltpu.SemaphoreType.DMA,
        ],
    )
    def k(x_hbm, pos_hbm, xs_hbm, pos_v, rows_v, sem):
        nc = 2
        wid = lax.axis_index("s") * nc + lax.axis_index("c")
        a0 = wid * APW
        t0 = lax.rem(a0, T)
        pltpu.sync_copy(pos_hbm.at[pl.ds(a0, APW)], pos_v)
        pltpu.sync_copy(x_hbm.at[pl.ds(t0, APW)], rows_v)
        pltpu.async_copy(rows_v, xs_hbm.at[pos_v], sem).wait()

    return k


def _sc_scatter_x(xf, pos_flat):
    return _sc_scatter_kernel()(xf, pos_flat)


@functools.lru_cache(maxsize=None)
def _sc_gather_kernel():
    mesh = plsc.VectorSubcoreMesh(core_axis_name="c", subcore_axis_name="s")

    @functools.partial(
        pl.kernel,
        out_type=(
            jax.ShapeDtypeStruct((T, D_MODEL // 2), jnp.int32),
            jax.ShapeDtypeStruct((T, D_MODEL // 2), jnp.int32),
        ),
        mesh=mesh,
        scratch_types=[
            pltpu.VMEM((TPW,), jnp.int32),
            pltpu.VMEM((TPW,), jnp.int32),
            pltpu.VMEM((TPW, D_MODEL // 2), jnp.int32),
            pltpu.VMEM((TPW, D_MODEL // 2), jnp.int32),
            pltpu.SemaphoreType.DMA,
            pltpu.SemaphoreType.DMA,
        ],
    )
    def k(ys_hbm, pos_hbm, ga_hbm, gb_hbm, p0_v, p1_v, a_v, b_v, s0, s1):
        nc = 2
        wid = lax.axis_index("s") * nc + lax.axis_index("c")
        t0 = wid * TPW
        pltpu.sync_copy(pos_hbm.at[pl.ds(t0, TPW)], p0_v)
        pltpu.sync_copy(pos_hbm.at[pl.ds(T + t0, TPW)], p1_v)
        ca = pltpu.async_copy(ys_hbm.at[p0_v], a_v, s0)
        cb = pltpu.async_copy(ys_hbm.at[p1_v], b_v, s1)
        ca.wait()
        cb.wait()
        pltpu.sync_copy(a_v, ga_hbm.at[pl.ds(t0, TPW)])
        pltpu.sync_copy(b_v, gb_hbm.at[pl.ds(t0, TPW)])

    return k


def _sc_gather_ys(ys, pos_flat):
    return _sc_gather_kernel()(ys, pos_flat)


def _mlp_body(be_ref, nv_ref, xs_ref, W1_ref, b1_ref, W2_ref, b2_ref, ys_ref):
    b = pl.program_id(0)

    @pl.when(b < nv_ref[0])
    def _():
        xs = _unpack_rows(xs_ref[...])
        h = jnp.dot(xs, W1_ref[0], preferred_element_type=jnp.float32)
        h = _gelu(h + b1_ref[0])
        y = jnp.dot(h, W2_ref[0], preferred_element_type=jnp.float32) + b2_ref[0]
        ys_ref[...] = _pack_rows(y)


def _combine_body(w_ref, ga_ref, gb_ref, out_ref):
    w = w_ref[...]
    ga = _unpack_rows(ga_ref[...])
    gb = _unpack_rows(gb_ref[...])
    out_ref[...] = ga * w[:, 0:1] + gb * w[:, 1:2]


def kernel(x, gate_w, W1, b1, W2, b2):
    B, S, D = x.shape
    xf = x.reshape(S, D)

    pos_c, w2c, xpk, be2, nv2 = pl.pallas_call(
        _routing_body,
        out_shape=(
            jax.ShapeDtypeStruct((A, 1), jnp.int32),
            jax.ShapeDtypeStruct((T, 2), jnp.float32),
            jax.ShapeDtypeStruct((T, _H), jnp.int32),
            jax.ShapeDtypeStruct((NB, 1), jnp.int32),
            jax.ShapeDtypeStruct((1, 1), jnp.int32),
        ),
    )(xf, gate_w)

    pos_flat = pos_c.reshape(A)
    be = be2[:, 0]
    nv = nv2[0]

    xs = _sc_scatter_x(xpk, pos_flat)

    ys = pl.pallas_call(
        _mlp_body,
        grid_spec=pltpu.PrefetchScalarGridSpec(
            num_scalar_prefetch=2,
            grid=(NB,),
            in_specs=[
                pl.BlockSpec((M, _H), lambda b, be, nv: (b, 0)),
                pl.BlockSpec((1, D_MODEL, D_FF), lambda b, be, nv: (be[b], 0, 0)),
                pl.BlockSpec((1, 1, D_FF), lambda b, be, nv: (be[b], 0, 0)),
                pl.BlockSpec((1, D_FF, D_MODEL), lambda b, be, nv: (be[b], 0, 0)),
                pl.BlockSpec((1, 1, D_MODEL), lambda b, be, nv: (be[b], 0, 0)),
            ],
            out_specs=pl.BlockSpec((M, _H), lambda b, be, nv: (b, 0)),
        ),
        out_shape=jax.ShapeDtypeStruct((P, _H), jnp.int32),
        compiler_params=pltpu.CompilerParams(
            dimension_semantics=("arbitrary",),
        ),
    )(be, nv, xs, W1, b1.reshape(E, 1, D_FF), W2, b2.reshape(E, 1, D_MODEL))

    ga, gb = _sc_gather_ys(ys, pos_flat)

    out = pl.pallas_call(
        _combine_body,
        out_shape=jax.ShapeDtypeStruct((T, D_MODEL), jnp.float32),
    )(w2c, ga, gb)

    return out.reshape(B, S, D)
